# baseline (speedup 1.0000x reference)
import functools

import jax
import jax.numpy as jnp
from jax.experimental import pallas as pl
from jax.experimental.pallas import tpu as pltpu

_NC = 2
_BT = 512
_SUB = 8


def _stats_kernel(x_ref, s1_ref, s2_ref):
    i = pl.program_id(1)
    bt, d_dim, _ = x_ref.shape

    s1_acc = None
    s2_acc = [None] * d_dim
    for c in range(bt // _SUB):
        xb = x_ref[c * _SUB:(c + 1) * _SUB, :, :]
        part = jnp.sum(xb, axis=0)
        s1_acc = part if s1_acc is None else s1_acc + part
        for d in range(d_dim):
            e0 = (d // _SUB) * _SUB
            prod = xb[:, e0:, :] * xb[:, d:d + 1, :]
            ps = jnp.sum(prod, axis=0)
            s2_acc[d] = ps if s2_acc[d] is None else s2_acc[d] + ps

    @pl.when(i == 0)
    def _():
        s1_ref[0] = s1_acc
        for d in range(d_dim):
            e0 = (d // _SUB) * _SUB
            s2_ref[0, d, e0:, :] = s2_acc[d]

    @pl.when(i != 0)
    def _():
        s1_ref[0] += s1_acc
        for d in range(d_dim):
            e0 = (d // _SUB) * _SUB
            s2_ref[0, d, e0:, :] += s2_acc[d]


def _factor_kernel(s1_ref, s2_ref, mu0_ref, l0t_ref, n0_ref,
                   wt_ref, nmu_ref, wmu_ref, m_ref, w_ref, *, bn):
    d_dim, k_dim = s1_ref.shape[1], s1_ref.shape[2]
    n0 = n0_ref[0]
    denom = n0 + bn
    scale_g = n0 / denom
    scale_c = 1.0 / denom
    scale_x = n0 * bn / (denom * denom)

    s1 = s1_ref[0] + s1_ref[1]
    s2u = s2_ref[0] + s2_ref[1]
    x_mu = s1 * (1.0 / bn)
    mu0 = mu0_ref[...]
    xd = x_mu - mu0
    nmu = (n0 * mu0 + bn * x_mu) / denom
    nmu_ref[...] = nmu

    l0 = l0t_ref[...]
    ei = jax.lax.broadcasted_iota(jnp.int32, (d_dim, k_dim), 0)

    for d in range(d_dim):
        e0 = (d // _SUB) * _SUB
        srow = jnp.where(ei >= e0, s2u[d], s2u[:, d, :])
        crow = srow - bn * (x_mu[d:d + 1, :] * x_mu)
        grow = jnp.sum(l0[d:d + 1, :, :] * l0, axis=1)
        eyed = jnp.where(ei == d, 1.0, 0.0)
        m_ref[d] = (grow * scale_g + crow * scale_c
                    + (xd[d:d + 1, :] * xd) * scale_x + eyed)

    for j in range(d_dim):
        dinv = jax.lax.rsqrt(m_ref[j, j:j + 1, :])
        col = m_ref[j:, j, :] * dinv
        m_ref[j:, j, :] = col
        if j + 1 < d_dim:
            sub = col[1:]
            m_ref[j + 1:, j + 1:, :] -= sub[:, None, :] * sub[None, :, :]

    for j in range(d_dim):
        rcp = 1.0 / m_ref[j, j:j + 1, :]
        ejrow = jnp.where(ei == j, 1.0, 0.0)
        if j == 0:
            wrow = ejrow * rcp
        else:
            rowl = m_ref[j, :j, :]
            acc = jnp.sum(rowl[:, None, :] * w_ref[:j, :, :], axis=0)
            wrow = (ejrow - acc) * rcp
        w_ref[j] = wrow
        wmu_ref[j:j + 1, :] = jnp.sum(wrow * nmu, axis=0, keepdims=True)

    for e in range(d_dim):
        wt_ref[e] = w_ref[:, e, :]


def _whiten_kernel(x_ref, wt_ref, wmu_ref, o_ref):
    bt, d_dim, _ = x_ref.shape
    ng = d_dim // _SUB
    wt = wt_ref[...]
    wmu = wmu_ref[...]
    for c in range(bt // _SUB):
        sl = slice(c * _SUB, (c + 1) * _SUB)
        accs = [None] * ng
        for e in range(d_dim):
            xbc = x_ref[sl, e:e + 1, :]
            for g in range(e // _SUB, ng):
                dsl = slice(g * _SUB, (g + 1) * _SUB)
                t = xbc * wt[e, dsl, :][None, :, :]
                accs[g] = t if accs[g] is None else accs[g] + t
        for g in range(ng):
            dsl = slice(g * _SUB, (g + 1) * _SUB)
            o_ref[sl, dsl, :] = accs[g] - wmu[dsl, :][None, :, :]


def kernel(x, mu_0, L_0, n_0):
    b, d_dim, k_dim = x.shape
    nb = b // (_NC * _BT)
    l0t = jnp.transpose(L_0, (1, 2, 0))

    s1p, s2p = pl.pallas_call(
        _stats_kernel,
        grid=(_NC, nb),
        in_specs=[
            pl.BlockSpec((_BT, d_dim, k_dim),
                         lambda p, i: (p * nb + i, 0, 0)),
        ],
        out_specs=[
            pl.BlockSpec((1, d_dim, k_dim), lambda p, i: (p, 0, 0)),
            pl.BlockSpec((1, d_dim, d_dim, k_dim), lambda p, i: (p, 0, 0, 0)),
        ],
        out_shape=[
            jax.ShapeDtypeStruct((_NC, d_dim, k_dim), jnp.float32),
            jax.ShapeDtypeStruct((_NC, d_dim, d_dim, k_dim), jnp.float32),
        ],
        compiler_params=pltpu.CompilerParams(
            dimension_semantics=("parallel", "arbitrary"),
        ),
        name="cluster_stats",
    )(x)

    wt, nmu, wmu = pl.pallas_call(
        functools.partial(_factor_kernel, bn=float(b)),
        in_specs=[
            pl.BlockSpec((_NC, d_dim, k_dim), lambda: (0, 0, 0)),
            pl.BlockSpec((_NC, d_dim, d_dim, k_dim), lambda: (0, 0, 0, 0)),
            pl.BlockSpec((d_dim, k_dim), lambda: (0, 0)),
            pl.BlockSpec((d_dim, d_dim, k_dim), lambda: (0, 0, 0)),
            pl.BlockSpec(memory_space=pltpu.SMEM),
        ],
        out_specs=[
            pl.BlockSpec((d_dim, d_dim, k_dim), lambda: (0, 0, 0)),
            pl.BlockSpec((d_dim, k_dim), lambda: (0, 0)),
            pl.BlockSpec((d_dim, k_dim), lambda: (0, 0)),
        ],
        out_shape=[
            jax.ShapeDtypeStruct((d_dim, d_dim, k_dim), jnp.float32),
            jax.ShapeDtypeStruct((d_dim, k_dim), jnp.float32),
            jax.ShapeDtypeStruct((d_dim, k_dim), jnp.float32),
        ],
        scratch_shapes=[
            pltpu.VMEM((d_dim, d_dim, k_dim), jnp.float32),
            pltpu.VMEM((d_dim, d_dim, k_dim), jnp.float32),
        ],
        name="cluster_factor",
    )(s1p, s2p, mu_0, l0t, n_0)

    z = pl.pallas_call(
        _whiten_kernel,
        grid=(_NC, nb),
        in_specs=[
            pl.BlockSpec((_BT, d_dim, k_dim),
                         lambda p, i: (p * nb + i, 0, 0)),
            pl.BlockSpec((d_dim, d_dim, k_dim), lambda p, i: (0, 0, 0)),
            pl.BlockSpec((d_dim, k_dim), lambda p, i: (0, 0)),
        ],
        out_specs=pl.BlockSpec((_BT, d_dim, k_dim),
                               lambda p, i: (p * nb + i, 0, 0)),
        out_shape=jax.ShapeDtypeStruct((b, d_dim, k_dim), jnp.float32),
        compiler_params=pltpu.CompilerParams(
            dimension_semantics=("parallel", "arbitrary"),
            vmem_limit_bytes=50 * 1024 * 1024,
        ),
        name="cluster_whiten",
    )(x, wt, wmu)
    return z


# speedup vs baseline: 1.0009x; 1.0009x over previous
import functools

import jax
import jax.numpy as jnp
from jax.experimental import pallas as pl
from jax.experimental.pallas import tpu as pltpu

_NC = 2
_BT = 256
_SUB = 8


def _stats_kernel(x_ref, s1_ref, s2_ref):
    i = pl.program_id(1)
    bt, d_dim, _ = x_ref.shape

    s1_acc = None
    s2_acc = [None] * d_dim
    for c in range(bt // _SUB):
        xb = x_ref[c * _SUB:(c + 1) * _SUB, :, :]
        part = jnp.sum(xb, axis=0)
        s1_acc = part if s1_acc is None else s1_acc + part
        for d in range(d_dim):
            e0 = (d // _SUB) * _SUB
            prod = xb[:, e0:, :] * xb[:, d:d + 1, :]
            ps = jnp.sum(prod, axis=0)
            s2_acc[d] = ps if s2_acc[d] is None else s2_acc[d] + ps

    @pl.when(i == 0)
    def _():
        s1_ref[0] = s1_acc
        for d in range(d_dim):
            e0 = (d // _SUB) * _SUB
            s2_ref[0, d, e0:, :] = s2_acc[d]

    @pl.when(i != 0)
    def _():
        s1_ref[0] += s1_acc
        for d in range(d_dim):
            e0 = (d // _SUB) * _SUB
            s2_ref[0, d, e0:, :] += s2_acc[d]


def _factor_kernel(s1_ref, s2_ref, mu0_ref, l0t_ref, n0_ref,
                   wt_ref, nmu_ref, wmu_ref, m_ref, w_ref, *, bn):
    d_dim, k_dim = s1_ref.shape[1], s1_ref.shape[2]
    n0 = n0_ref[0]
    denom = n0 + bn
    scale_g = n0 / denom
    scale_c = 1.0 / denom
    scale_x = n0 * bn / (denom * denom)

    s1 = s1_ref[0] + s1_ref[1]
    s2u = s2_ref[0] + s2_ref[1]
    x_mu = s1 * (1.0 / bn)
    mu0 = mu0_ref[...]
    xd = x_mu - mu0
    nmu = (n0 * mu0 + bn * x_mu) / denom
    nmu_ref[...] = nmu

    l0 = l0t_ref[...]
    ei = jax.lax.broadcasted_iota(jnp.int32, (d_dim, k_dim), 0)

    for d in range(d_dim):
        e0 = (d // _SUB) * _SUB
        srow = jnp.where(ei >= e0, s2u[d], s2u[:, d, :])
        crow = srow - bn * (x_mu[d:d + 1, :] * x_mu)
        grow = jnp.sum(l0[d:d + 1, :, :] * l0, axis=1)
        eyed = jnp.where(ei == d, 1.0, 0.0)
        m_ref[d] = (grow * scale_g + crow * scale_c
                    + (xd[d:d + 1, :] * xd) * scale_x + eyed)

    for j in range(d_dim):
        dinv = jax.lax.rsqrt(m_ref[j, j:j + 1, :])
        col = m_ref[j:, j, :] * dinv
        m_ref[j:, j, :] = col
        if j + 1 < d_dim:
            sub = col[1:]
            m_ref[j + 1:, j + 1:, :] -= sub[:, None, :] * sub[None, :, :]

    for j in range(d_dim):
        rcp = 1.0 / m_ref[j, j:j + 1, :]
        ejrow = jnp.where(ei == j, 1.0, 0.0)
        if j == 0:
            wrow = ejrow * rcp
        else:
            rowl = m_ref[j, :j, :]
            acc = jnp.sum(rowl[:, None, :] * w_ref[:j, :, :], axis=0)
            wrow = (ejrow - acc) * rcp
        w_ref[j] = wrow
        wmu_ref[j:j + 1, :] = jnp.sum(wrow * nmu, axis=0, keepdims=True)

    for e in range(d_dim):
        wt_ref[e] = w_ref[:, e, :]


def _whiten_kernel(x_ref, wt_ref, wmu_ref, o_ref):
    bt, d_dim, _ = x_ref.shape
    ng = d_dim // _SUB
    wt = wt_ref[...]
    wmu = wmu_ref[...]
    for c in range(bt // _SUB):
        sl = slice(c * _SUB, (c + 1) * _SUB)
        accs = [None] * ng
        for e in range(d_dim):
            xbc = x_ref[sl, e:e + 1, :]
            for g in range(e // _SUB, ng):
                dsl = slice(g * _SUB, (g + 1) * _SUB)
                t = xbc * wt[e, dsl, :][None, :, :]
                accs[g] = t if accs[g] is None else accs[g] + t
        for g in range(ng):
            dsl = slice(g * _SUB, (g + 1) * _SUB)
            o_ref[sl, dsl, :] = accs[g] - wmu[dsl, :][None, :, :]


def kernel(x, mu_0, L_0, n_0):
    b, d_dim, k_dim = x.shape
    nb = b // (_NC * _BT)
    l0t = jnp.transpose(L_0, (1, 2, 0))

    s1p, s2p = pl.pallas_call(
        _stats_kernel,
        grid=(_NC, nb),
        in_specs=[
            pl.BlockSpec((_BT, d_dim, k_dim),
                         lambda p, i: (p * nb + i, 0, 0)),
        ],
        out_specs=[
            pl.BlockSpec((1, d_dim, k_dim), lambda p, i: (p, 0, 0)),
            pl.BlockSpec((1, d_dim, d_dim, k_dim), lambda p, i: (p, 0, 0, 0)),
        ],
        out_shape=[
            jax.ShapeDtypeStruct((_NC, d_dim, k_dim), jnp.float32),
            jax.ShapeDtypeStruct((_NC, d_dim, d_dim, k_dim), jnp.float32),
        ],
        compiler_params=pltpu.CompilerParams(
            dimension_semantics=("parallel", "arbitrary"),
        ),
        name="cluster_stats",
    )(x)

    wt, nmu, wmu = pl.pallas_call(
        functools.partial(_factor_kernel, bn=float(b)),
        in_specs=[
            pl.BlockSpec((_NC, d_dim, k_dim), lambda: (0, 0, 0)),
            pl.BlockSpec((_NC, d_dim, d_dim, k_dim), lambda: (0, 0, 0, 0)),
            pl.BlockSpec((d_dim, k_dim), lambda: (0, 0)),
            pl.BlockSpec((d_dim, d_dim, k_dim), lambda: (0, 0, 0)),
            pl.BlockSpec(memory_space=pltpu.SMEM),
        ],
        out_specs=[
            pl.BlockSpec((d_dim, d_dim, k_dim), lambda: (0, 0, 0)),
            pl.BlockSpec((d_dim, k_dim), lambda: (0, 0)),
            pl.BlockSpec((d_dim, k_dim), lambda: (0, 0)),
        ],
        out_shape=[
            jax.ShapeDtypeStruct((d_dim, d_dim, k_dim), jnp.float32),
            jax.ShapeDtypeStruct((d_dim, k_dim), jnp.float32),
            jax.ShapeDtypeStruct((d_dim, k_dim), jnp.float32),
        ],
        scratch_shapes=[
            pltpu.VMEM((d_dim, d_dim, k_dim), jnp.float32),
            pltpu.VMEM((d_dim, d_dim, k_dim), jnp.float32),
        ],
        name="cluster_factor",
    )(s1p, s2p, mu_0, l0t, n_0)

    z = pl.pallas_call(
        _whiten_kernel,
        grid=(_NC, nb),
        in_specs=[
            pl.BlockSpec((_BT, d_dim, k_dim),
                         lambda p, i: (p * nb + i, 0, 0)),
            pl.BlockSpec((d_dim, d_dim, k_dim), lambda p, i: (0, 0, 0)),
            pl.BlockSpec((d_dim, k_dim), lambda p, i: (0, 0)),
        ],
        out_specs=pl.BlockSpec((_BT, d_dim, k_dim),
                               lambda p, i: (p * nb + i, 0, 0)),
        out_shape=jax.ShapeDtypeStruct((b, d_dim, k_dim), jnp.float32),
        compiler_params=pltpu.CompilerParams(
            dimension_semantics=("parallel", "arbitrary"),
            vmem_limit_bytes=50 * 1024 * 1024,
        ),
        name="cluster_whiten",
    )(x, wt, wmu)
    return z


# speedup vs baseline: 1.0344x; 1.0334x over previous
import functools

import jax
import jax.numpy as jnp
from jax.experimental import pallas as pl
from jax.experimental.pallas import tpu as pltpu

_NC = 2
_BT = 256
_SUB = 8


def _stats_kernel(x_ref, s1_ref, s2_ref):
    i = pl.program_id(1)
    bt, d_dim, _ = x_ref.shape

    s1_acc = None
    s2_acc = [None] * d_dim
    for c in range(bt // _SUB):
        xb = x_ref[c * _SUB:(c + 1) * _SUB, :, :]
        part = jnp.sum(xb, axis=0)
        s1_acc = part if s1_acc is None else s1_acc + part
        xbf = xb.astype(jnp.bfloat16)
        for d in range(d_dim):
            e0 = (d // _SUB) * _SUB
            prod = xbf[:, e0:, :] * xbf[:, d:d + 1, :]
            t = prod[0:4] + prod[4:8]
            t = t[0:2] + t[2:4]
            ps = (t[0] + t[1]).astype(jnp.float32)
            s2_acc[d] = ps if s2_acc[d] is None else s2_acc[d] + ps

    @pl.when(i == 0)
    def _():
        s1_ref[0] = s1_acc
        for d in range(d_dim):
            e0 = (d // _SUB) * _SUB
            s2_ref[0, d, e0:, :] = s2_acc[d]

    @pl.when(i != 0)
    def _():
        s1_ref[0] += s1_acc
        for d in range(d_dim):
            e0 = (d // _SUB) * _SUB
            s2_ref[0, d, e0:, :] += s2_acc[d]


def _factor_kernel(s1_ref, s2_ref, mu0_ref, l0t_ref, n0_ref,
                   wt_ref, nmu_ref, wmu_ref, m_ref, w_ref, *, bn):
    d_dim, k_dim = s1_ref.shape[1], s1_ref.shape[2]
    n0 = n0_ref[0]
    denom = n0 + bn
    scale_g = n0 / denom
    scale_c = 1.0 / denom
    scale_x = n0 * bn / (denom * denom)

    s1 = s1_ref[0] + s1_ref[1]
    s2u = s2_ref[0] + s2_ref[1]
    x_mu = s1 * (1.0 / bn)
    mu0 = mu0_ref[...]
    xd = x_mu - mu0
    nmu = (n0 * mu0 + bn * x_mu) / denom
    nmu_ref[...] = nmu

    l0 = l0t_ref[...]
    ei = jax.lax.broadcasted_iota(jnp.int32, (d_dim, k_dim), 0)

    for d in range(d_dim):
        e0 = (d // _SUB) * _SUB
        srow = jnp.where(ei >= e0, s2u[d], s2u[:, d, :])
        crow = srow - bn * (x_mu[d:d + 1, :] * x_mu)
        grow = jnp.sum(l0[d:d + 1, :, :] * l0, axis=1)
        eyed = jnp.where(ei == d, 1.0, 0.0)
        m_ref[d] = (grow * scale_g + crow * scale_c
                    + (xd[d:d + 1, :] * xd) * scale_x + eyed)

    for j in range(d_dim):
        dinv = jax.lax.rsqrt(m_ref[j, j:j + 1, :])
        col = m_ref[j:, j, :] * dinv
        m_ref[j:, j, :] = col
        if j + 1 < d_dim:
            sub = col[1:]
            m_ref[j + 1:, j + 1:, :] -= sub[:, None, :] * sub[None, :, :]

    for j in range(d_dim):
        rcp = 1.0 / m_ref[j, j:j + 1, :]
        ejrow = jnp.where(ei == j, 1.0, 0.0)
        if j == 0:
            wrow = ejrow * rcp
        else:
            rowl = m_ref[j, :j, :]
            acc = jnp.sum(rowl[:, None, :] * w_ref[:j, :, :], axis=0)
            wrow = (ejrow - acc) * rcp
        w_ref[j] = wrow
        wmu_ref[j:j + 1, :] = jnp.sum(wrow * nmu, axis=0, keepdims=True)

    for e in range(d_dim):
        wt_ref[e] = w_ref[:, e, :]


def _whiten_kernel(x_ref, wt_ref, wmu_ref, o_ref):
    bt, d_dim, _ = x_ref.shape
    ng = d_dim // _SUB
    wt = wt_ref[...]
    wmu = wmu_ref[...]
    for c in range(bt // _SUB):
        sl = slice(c * _SUB, (c + 1) * _SUB)
        accs = [None] * ng
        for e in range(d_dim):
            xbc = x_ref[sl, e:e + 1, :]
            for g in range(e // _SUB, ng):
                dsl = slice(g * _SUB, (g + 1) * _SUB)
                t = xbc * wt[e, dsl, :][None, :, :]
                accs[g] = t if accs[g] is None else accs[g] + t
        for g in range(ng):
            dsl = slice(g * _SUB, (g + 1) * _SUB)
            o_ref[sl, dsl, :] = accs[g] - wmu[dsl, :][None, :, :]


def kernel(x, mu_0, L_0, n_0):
    b, d_dim, k_dim = x.shape
    nb = b // (_NC * _BT)
    l0t = jnp.transpose(L_0, (1, 2, 0))

    s1p, s2p = pl.pallas_call(
        _stats_kernel,
        grid=(_NC, nb),
        in_specs=[
            pl.BlockSpec((_BT, d_dim, k_dim),
                         lambda p, i: (p * nb + i, 0, 0)),
        ],
        out_specs=[
            pl.BlockSpec((1, d_dim, k_dim), lambda p, i: (p, 0, 0)),
            pl.BlockSpec((1, d_dim, d_dim, k_dim), lambda p, i: (p, 0, 0, 0)),
        ],
        out_shape=[
            jax.ShapeDtypeStruct((_NC, d_dim, k_dim), jnp.float32),
            jax.ShapeDtypeStruct((_NC, d_dim, d_dim, k_dim), jnp.float32),
        ],
        compiler_params=pltpu.CompilerParams(
            dimension_semantics=("parallel", "arbitrary"),
        ),
        name="cluster_stats",
    )(x)

    wt, nmu, wmu = pl.pallas_call(
        functools.partial(_factor_kernel, bn=float(b)),
        in_specs=[
            pl.BlockSpec((_NC, d_dim, k_dim), lambda: (0, 0, 0)),
            pl.BlockSpec((_NC, d_dim, d_dim, k_dim), lambda: (0, 0, 0, 0)),
            pl.BlockSpec((d_dim, k_dim), lambda: (0, 0)),
            pl.BlockSpec((d_dim, d_dim, k_dim), lambda: (0, 0, 0)),
            pl.BlockSpec(memory_space=pltpu.SMEM),
        ],
        out_specs=[
            pl.BlockSpec((d_dim, d_dim, k_dim), lambda: (0, 0, 0)),
            pl.BlockSpec((d_dim, k_dim), lambda: (0, 0)),
            pl.BlockSpec((d_dim, k_dim), lambda: (0, 0)),
        ],
        out_shape=[
            jax.ShapeDtypeStruct((d_dim, d_dim, k_dim), jnp.float32),
            jax.ShapeDtypeStruct((d_dim, k_dim), jnp.float32),
            jax.ShapeDtypeStruct((d_dim, k_dim), jnp.float32),
        ],
        scratch_shapes=[
            pltpu.VMEM((d_dim, d_dim, k_dim), jnp.float32),
            pltpu.VMEM((d_dim, d_dim, k_dim), jnp.float32),
        ],
        name="cluster_factor",
    )(s1p, s2p, mu_0, l0t, n_0)

    z = pl.pallas_call(
        _whiten_kernel,
        grid=(_NC, nb),
        in_specs=[
            pl.BlockSpec((_BT, d_dim, k_dim),
                         lambda p, i: (p * nb + i, 0, 0)),
            pl.BlockSpec((d_dim, d_dim, k_dim), lambda p, i: (0, 0, 0)),
            pl.BlockSpec((d_dim, k_dim), lambda p, i: (0, 0)),
        ],
        out_specs=pl.BlockSpec((_BT, d_dim, k_dim),
                               lambda p, i: (p * nb + i, 0, 0)),
        out_shape=jax.ShapeDtypeStruct((b, d_dim, k_dim), jnp.float32),
        compiler_params=pltpu.CompilerParams(
            dimension_semantics=("parallel", "arbitrary"),
            vmem_limit_bytes=50 * 1024 * 1024,
        ),
        name="cluster_whiten",
    )(x, wt, wmu)
    return z


# speedup vs baseline: 1.0463x; 1.0116x over previous
import functools

import jax
import jax.numpy as jnp
from jax.experimental import pallas as pl
from jax.experimental.pallas import tpu as pltpu

_NC = 2
_BT = 256
_SUB = 8


def _stats_kernel(x_ref, s1_ref, s2_ref):
    i = pl.program_id(1)
    bt, d_dim, _ = x_ref.shape

    s1_acc = None
    s2_acc = [None] * d_dim
    for c in range(bt // (2 * _SUB)):
        parts = [None, None]
        for h in range(2):
            c8 = 2 * c + h
            xb = x_ref[c8 * _SUB:(c8 + 1) * _SUB, :, :]
            part = jnp.sum(xb, axis=0)
            s1_acc = part if s1_acc is None else s1_acc + part
            xbf = xb.astype(jnp.bfloat16)
            for d in range(d_dim):
                e0 = (d // _SUB) * _SUB
                prod = xbf[:, e0:, :] * xbf[:, d:d + 1, :]
                t = prod[0:4] + prod[4:8]
                t = t[0:2] + t[2:4]
                parts[h] = parts[h] or [None] * d_dim
                parts[h][d] = t[0] + t[1]
        for d in range(d_dim):
            ps = (parts[0][d] + parts[1][d]).astype(jnp.float32)
            s2_acc[d] = ps if s2_acc[d] is None else s2_acc[d] + ps

    @pl.when(i == 0)
    def _():
        s1_ref[0] = s1_acc
        for d in range(d_dim):
            e0 = (d // _SUB) * _SUB
            s2_ref[0, d, e0:, :] = s2_acc[d]

    @pl.when(i != 0)
    def _():
        s1_ref[0] += s1_acc
        for d in range(d_dim):
            e0 = (d // _SUB) * _SUB
            s2_ref[0, d, e0:, :] += s2_acc[d]


def _factor_kernel(s1_ref, s2_ref, mu0_ref, l0t_ref, n0_ref,
                   wt_ref, nmu_ref, wmu_ref, m_ref, w_ref, *, bn):
    d_dim, k_dim = s1_ref.shape[1], s1_ref.shape[2]
    n0 = n0_ref[0]
    denom = n0 + bn
    scale_g = n0 / denom
    scale_c = 1.0 / denom
    scale_x = n0 * bn / (denom * denom)

    s1 = s1_ref[0] + s1_ref[1]
    s2u = s2_ref[0] + s2_ref[1]
    x_mu = s1 * (1.0 / bn)
    mu0 = mu0_ref[...]
    xd = x_mu - mu0
    nmu = (n0 * mu0 + bn * x_mu) / denom
    nmu_ref[...] = nmu

    l0 = l0t_ref[...]
    ei = jax.lax.broadcasted_iota(jnp.int32, (d_dim, k_dim), 0)

    for d in range(d_dim):
        e0 = (d // _SUB) * _SUB
        srow = jnp.where(ei >= e0, s2u[d], s2u[:, d, :])
        crow = srow - bn * (x_mu[d:d + 1, :] * x_mu)
        grow = jnp.sum(l0[d:d + 1, :, :] * l0, axis=1)
        eyed = jnp.where(ei == d, 1.0, 0.0)
        m_ref[d] = (grow * scale_g + crow * scale_c
                    + (xd[d:d + 1, :] * xd) * scale_x + eyed)

    for j in range(d_dim):
        dinv = jax.lax.rsqrt(m_ref[j, j:j + 1, :])
        col = m_ref[j:, j, :] * dinv
        m_ref[j:, j, :] = col
        if j + 1 < d_dim:
            sub = col[1:]
            m_ref[j + 1:, j + 1:, :] -= sub[:, None, :] * sub[None, :, :]

    for j in range(d_dim):
        rcp = 1.0 / m_ref[j, j:j + 1, :]
        ejrow = jnp.where(ei == j, 1.0, 0.0)
        if j == 0:
            wrow = ejrow * rcp
        else:
            rowl = m_ref[j, :j, :]
            acc = jnp.sum(rowl[:, None, :] * w_ref[:j, :, :], axis=0)
            wrow = (ejrow - acc) * rcp
        w_ref[j] = wrow
        wmu_ref[j:j + 1, :] = jnp.sum(wrow * nmu, axis=0, keepdims=True)

    for e in range(d_dim):
        wt_ref[e] = w_ref[:, e, :]


def _whiten_kernel(x_ref, wt_ref, wmu_ref, o_ref):
    bt, d_dim, _ = x_ref.shape
    ng = d_dim // _SUB
    wt = wt_ref[...]
    wmu = wmu_ref[...]
    for c in range(bt // _SUB):
        sl = slice(c * _SUB, (c + 1) * _SUB)
        accs = [None] * ng
        for e in range(d_dim):
            xbc = x_ref[sl, e:e + 1, :]
            for g in range(e // _SUB, ng):
                dsl = slice(g * _SUB, (g + 1) * _SUB)
                t = xbc * wt[e, dsl, :][None, :, :]
                accs[g] = t if accs[g] is None else accs[g] + t
        for g in range(ng):
            dsl = slice(g * _SUB, (g + 1) * _SUB)
            o_ref[sl, dsl, :] = accs[g] - wmu[dsl, :][None, :, :]


def kernel(x, mu_0, L_0, n_0):
    b, d_dim, k_dim = x.shape
    nb = b // (_NC * _BT)
    l0t = jnp.transpose(L_0, (1, 2, 0))

    s1p, s2p = pl.pallas_call(
        _stats_kernel,
        grid=(_NC, nb),
        in_specs=[
            pl.BlockSpec((_BT, d_dim, k_dim),
                         lambda p, i: (p * nb + i, 0, 0)),
        ],
        out_specs=[
            pl.BlockSpec((1, d_dim, k_dim), lambda p, i: (p, 0, 0)),
            pl.BlockSpec((1, d_dim, d_dim, k_dim), lambda p, i: (p, 0, 0, 0)),
        ],
        out_shape=[
            jax.ShapeDtypeStruct((_NC, d_dim, k_dim), jnp.float32),
            jax.ShapeDtypeStruct((_NC, d_dim, d_dim, k_dim), jnp.float32),
        ],
        compiler_params=pltpu.CompilerParams(
            dimension_semantics=("parallel", "arbitrary"),
        ),
        name="cluster_stats",
    )(x)

    wt, nmu, wmu = pl.pallas_call(
        functools.partial(_factor_kernel, bn=float(b)),
        in_specs=[
            pl.BlockSpec((_NC, d_dim, k_dim), lambda: (0, 0, 0)),
            pl.BlockSpec((_NC, d_dim, d_dim, k_dim), lambda: (0, 0, 0, 0)),
            pl.BlockSpec((d_dim, k_dim), lambda: (0, 0)),
            pl.BlockSpec((d_dim, d_dim, k_dim), lambda: (0, 0, 0)),
            pl.BlockSpec(memory_space=pltpu.SMEM),
        ],
        out_specs=[
            pl.BlockSpec((d_dim, d_dim, k_dim), lambda: (0, 0, 0)),
            pl.BlockSpec((d_dim, k_dim), lambda: (0, 0)),
            pl.BlockSpec((d_dim, k_dim), lambda: (0, 0)),
        ],
        out_shape=[
            jax.ShapeDtypeStruct((d_dim, d_dim, k_dim), jnp.float32),
            jax.ShapeDtypeStruct((d_dim, k_dim), jnp.float32),
            jax.ShapeDtypeStruct((d_dim, k_dim), jnp.float32),
        ],
        scratch_shapes=[
            pltpu.VMEM((d_dim, d_dim, k_dim), jnp.float32),
            pltpu.VMEM((d_dim, d_dim, k_dim), jnp.float32),
        ],
        name="cluster_factor",
    )(s1p, s2p, mu_0, l0t, n_0)

    z = pl.pallas_call(
        _whiten_kernel,
        grid=(_NC, nb),
        in_specs=[
            pl.BlockSpec((_BT, d_dim, k_dim),
                         lambda p, i: (p * nb + i, 0, 0)),
            pl.BlockSpec((d_dim, d_dim, k_dim), lambda p, i: (0, 0, 0)),
            pl.BlockSpec((d_dim, k_dim), lambda p, i: (0, 0)),
        ],
        out_specs=pl.BlockSpec((_BT, d_dim, k_dim),
                               lambda p, i: (p * nb + i, 0, 0)),
        out_shape=jax.ShapeDtypeStruct((b, d_dim, k_dim), jnp.float32),
        compiler_params=pltpu.CompilerParams(
            dimension_semantics=("parallel", "arbitrary"),
            vmem_limit_bytes=50 * 1024 * 1024,
        ),
        name="cluster_whiten",
    )(x, wt, wmu)
    return z
